# baseline (speedup 1.0000x reference)
import functools

import jax
import jax.numpy as jnp
from jax import lax
from jax.experimental import pallas as pl
from jax.experimental.pallas import tpu as pltpu
from jax.experimental.pallas import tpu_sc as plsc

_D = 5
_B = 16384
_H = 200
_TOTAL = _B * _H
_NW = 32
_V = 1000000
_RW = 40
_VR = _V * _D // _RW
_NT_H = _H // 8
_NT_B = _B // 128
_CL = 1024
_CPH = _NT_B * 1024 // (_CL * _NW)
_ROW_W = _NT_B * 1024
_PLANE = _NT_H * _ROW_W
_NCH = _NT_H * _CPH

_mesh = plsc.VectorSubcoreMesh(core_axis_name="c", subcore_axis_name="s")

_CKR = 2000
_NCHT = _V // _CKR


@functools.partial(
    pl.kernel,
    mesh=_mesh,
    out_type=jax.ShapeDtypeStruct((_V * _D,), jnp.float32),
    scratch_types=[
        pltpu.VMEM((_CKR, _D), jnp.float32),
        pltpu.VMEM((_CKR * _D,), jnp.float32),
    ],
    compiler_params=pltpu.CompilerParams(
        use_tc_tiling_on_sc=False, needs_layout_passes=False
    ),
)
def _compact_table(t_hbm, out_hbm, v5, vout):
    wid = lax.axis_index("s") * 2 + lax.axis_index("c")
    lane = lax.iota(jnp.int32, 16)
    pats = []
    for p in range(_D):
        t = p * 16 + lane
        pats.append((lax.div(t, _D), lax.rem(t, _D)))
    nch = jnp.where(wid < _NCHT % _NW, _NCHT // _NW + 1, _NCHT // _NW)

    def body(i, carry):
        c = i * _NW + wid
        pltpu.sync_copy(t_hbm.at[pl.ds(c * _CKR, _CKR)], v5)

        def gbody(g, cc):
            for p in range(_D):
                rowp, colp = pats[p]
                v = plsc.load_gather(v5, [g * 16 + rowp, colp])
                vout[pl.ds(g * 80 + p * 16, 16)] = v
            return cc

        lax.fori_loop(0, _CKR // 16, gbody, 0)
        pltpu.sync_copy(vout, out_hbm.at[pl.ds(c * _CKR * _D, _CKR * _D)])
        return carry

    lax.fori_loop(0, nch, body, 0)


@functools.partial(
    pl.kernel,
    mesh=_mesh,
    out_type=jax.ShapeDtypeStruct((_D * _TOTAL,), jnp.float32),
    scratch_types=[
        pltpu.VMEM((2, _CL), jnp.int32),
        pltpu.VMEM((2, _CL), jnp.int32),
        pltpu.VMEM((2, _CL, _RW), jnp.float32),
        pltpu.VMEM((2 * _D, _CL), jnp.float32),
        pltpu.SemaphoreType.DMA,
        pltpu.SemaphoreType.DMA,
    ],
    compiler_params=pltpu.CompilerParams(
        use_tc_tiling_on_sc=False, needs_layout_passes=False
    ),
)
def _embed_gather_t(idx_hbm, table_hbm, out_hbm, idx_v, row_v, rows_v, planes_v,
                    sem_g, sem_w):
    wid = lax.axis_index("s") * 2 + lax.axis_index("c")
    lane = lax.iota(jnp.int32, 16)

    def chunk_off(i):
        ht = i // _CPH
        q = i % _CPH
        return ht * _ROW_W + (wid * _CPH + q) * _CL

    def start_gather(i, b):
        off = chunk_off(i)
        pltpu.sync_copy(idx_hbm.at[pl.ds(off, _CL)], idx_v.at[b])

        def rbody(j, c):
            w16 = idx_v[b, pl.ds(j * 16, 16)]
            row_v[b, pl.ds(j * 16, 16)] = lax.shift_right_logical(w16, 3)
            return c

        lax.fori_loop(0, _CL // 16, rbody, 0)
        return pltpu.async_copy(table_hbm.at[row_v.at[b]], rows_v.at[b], sem_g)

    def extract(i, b):
        bb = jnp.full((16,), b, jnp.int32)

        def tbody(j, c):
            w16 = idx_v[b, pl.ds(j * 16, 16)]
            col = jnp.bitwise_and(w16, 7) * _D
            row16 = j * 16 + lane
            for d in range(_D):
                v = plsc.load_gather(rows_v, [bb, row16, col + d])
                planes_v[b * _D + d, pl.ds(j * 16, 16)] = v
            return c

        lax.fori_loop(0, _CL // 16, tbody, 0)
        off = chunk_off(i)
        return [
            pltpu.async_copy(
                planes_v.at[b * _D + d],
                out_hbm.at[pl.ds(d * _PLANE + off, _CL)],
                sem_w,
            )
            for d in range(_D)
        ]

    def body(p, carry):
        c0 = start_gather(2 * p, 0)
        c1 = start_gather(2 * p + 1, 1)
        c0.wait()
        w0 = extract(2 * p, 0)
        c1.wait()
        w1 = extract(2 * p + 1, 1)
        for w in w0 + w1:
            w.wait()
        return carry

    lax.fori_loop(0, _NCH // 2, body, 0)


def kernel(data_index, embedding_dim, table5):
    flag = jnp.asarray(embedding_dim == _D, jnp.int32)
    idx_t = (
        data_index.T.reshape(_NT_H, 8, _NT_B, 128)
        .transpose(0, 2, 1, 3)
        .reshape(_TOTAL)
    ) * flag
    table40 = _compact_table(table5).reshape(_VR, _RW)
    flat = _embed_gather_t(idx_t, table40)
    o5 = flat.reshape(_D, _NT_H, _NT_B, 8, 128).transpose(1, 3, 2, 4, 0)
    return o5.reshape(_H, _B, _D).transpose(1, 0, 2)


# speedup vs baseline: 1.3539x; 1.3539x over previous
import functools

import jax
import jax.numpy as jnp
from jax import lax
from jax.experimental import pallas as pl
from jax.experimental.pallas import tpu as pltpu
from jax.experimental.pallas import tpu_sc as plsc

_D = 5
_B = 16384
_H = 200
_TOTAL = _B * _H
_NW = 32
_V = 1000000
_RW = 16
_VR = _V * _D // _RW
_NT_H = _H // 8
_NT_B = _B // 128
_CL = 1024
_CPH = _NT_B * 1024 // (_CL * _NW)
_ROW_W = _NT_B * 1024
_PLANE = _NT_H * _ROW_W
_NCH = _NT_H * _CPH

_mesh = plsc.VectorSubcoreMesh(core_axis_name="c", subcore_axis_name="s")

@functools.partial(
    pl.kernel,
    mesh=_mesh,
    out_type=jax.ShapeDtypeStruct((_D * _TOTAL,), jnp.float32),
    scratch_types=[
        pltpu.VMEM((2, _CL), jnp.int32),
        pltpu.VMEM((4, _CL), jnp.int32),
        pltpu.VMEM((4, _CL, _RW), jnp.float32),
        pltpu.VMEM((2 * _D, _CL), jnp.float32),
        pltpu.SemaphoreType.DMA,
        pltpu.SemaphoreType.DMA,
    ],
    compiler_params=pltpu.CompilerParams(
        use_tc_tiling_on_sc=False, needs_layout_passes=False
    ),
)
def _embed_gather_t(idx_hbm, table_hbm, out_hbm, idx_v, row_v, rows_v, planes_v,
                    sem_g, sem_w):
    wid = lax.axis_index("s") * 2 + lax.axis_index("c")
    lane = lax.iota(jnp.int32, 16)

    def chunk_off(i):
        ht = i // _CPH
        q = i % _CPH
        return ht * _ROW_W + (wid * _CPH + q) * _CL

    def start_gather(i, b):
        off = chunk_off(i)
        pltpu.sync_copy(idx_hbm.at[pl.ds(off, _CL)], idx_v.at[b])

        def rbody(j, c):
            w16 = idx_v[b, pl.ds(j * 16, 16)] * _D
            ra = lax.shift_right_logical(w16, 4)
            row_v[2 * b, pl.ds(j * 16, 16)] = ra
            row_v[2 * b + 1, pl.ds(j * 16, 16)] = jnp.minimum(ra + 1, _VR - 1)
            return c

        lax.fori_loop(0, _CL // 16, rbody, 0)
        ca = pltpu.async_copy(
            table_hbm.at[row_v.at[2 * b]], rows_v.at[2 * b], sem_g)
        cb = pltpu.async_copy(
            table_hbm.at[row_v.at[2 * b + 1]], rows_v.at[2 * b + 1], sem_g)
        return ca, cb

    def extract(i, b):
        def tbody(j, c):
            w16 = jnp.bitwise_and(idx_v[b, pl.ds(j * 16, 16)] * _D, 15)
            row16 = j * 16 + lane
            for d in range(_D):
                off = w16 + d
                buf = 2 * b + lax.shift_right_logical(off, 4)
                v = plsc.load_gather(
                    rows_v, [buf, row16, jnp.bitwise_and(off, 15)])
                planes_v[b * _D + d, pl.ds(j * 16, 16)] = v
            return c

        lax.fori_loop(0, _CL // 16, tbody, 0)
        off = chunk_off(i)
        return [
            pltpu.async_copy(
                planes_v.at[b * _D + d],
                out_hbm.at[pl.ds(d * _PLANE + off, _CL)],
                sem_w,
            )
            for d in range(_D)
        ]

    def body(p, carry):
        c0a, c0b = start_gather(2 * p, 0)
        c1a, c1b = start_gather(2 * p + 1, 1)
        c0a.wait()
        c0b.wait()
        w0 = extract(2 * p, 0)
        c1a.wait()
        c1b.wait()
        w1 = extract(2 * p + 1, 1)
        for w in w0 + w1:
            w.wait()
        return carry

    lax.fori_loop(0, _NCH // 2, body, 0)


def kernel(data_index, embedding_dim, table5):
    flag = jnp.asarray(embedding_dim == _D, jnp.int32)
    idx_t = (
        data_index.T.reshape(_NT_H, 8, _NT_B, 128)
        .transpose(0, 2, 1, 3)
        .reshape(_TOTAL)
    ) * flag
    table40 = table5.reshape(_VR, _RW)
    flat = _embed_gather_t(idx_t, table40)
    o5 = flat.reshape(_D, _NT_H, _NT_B, 8, 128).transpose(1, 3, 2, 4, 0)
    return o5.reshape(_H, _B, _D).transpose(1, 0, 2)


# speedup vs baseline: 1.4413x; 1.0646x over previous
import functools

import jax
import jax.numpy as jnp
from jax import lax
from jax.experimental import pallas as pl
from jax.experimental.pallas import tpu as pltpu
from jax.experimental.pallas import tpu_sc as plsc

_D = 5
_B = 16384
_H = 200
_TOTAL = _B * _H
_NW = 32
_V = 1000000
_RW = 40
_VR = _V * _D // _RW
_NT_H = _H // 8
_NT_B = _B // 128
_CL = 512
_NB = 4
_CPH = _NT_B * 1024 // (_CL * _NW)
_ROW_W = _NT_B * 1024
_PLANE = _NT_H * _ROW_W
_NCH = _NT_H * _CPH

_mesh = plsc.VectorSubcoreMesh(core_axis_name="c", subcore_axis_name="s")

@functools.partial(
    pl.kernel,
    mesh=_mesh,
    out_type=jax.ShapeDtypeStruct((_D * _TOTAL,), jnp.float32),
    scratch_types=[
        pltpu.VMEM((_NB, _CL), jnp.int32),
        pltpu.VMEM((_NB, _CL), jnp.int32),
        pltpu.VMEM((_NB, _CL, _RW), jnp.float32),
        pltpu.VMEM((_NB * _D, _CL), jnp.float32),
        pltpu.SemaphoreType.DMA,
        pltpu.SemaphoreType.DMA,
    ],
    compiler_params=pltpu.CompilerParams(
        use_tc_tiling_on_sc=False, needs_layout_passes=False
    ),
)
def _embed_gather_t(idx_hbm, table_hbm, out_hbm, idx_v, row_v, rows_v, planes_v,
                    sem_g, sem_w):
    wid = lax.axis_index("s") * 2 + lax.axis_index("c")
    lane = lax.iota(jnp.int32, 16)

    def chunk_off(i):
        ht = i // _CPH
        q = i % _CPH
        return ht * _ROW_W + (wid * _CPH + q) * _CL

    def start_gather(i, b):
        off = chunk_off(i)
        pltpu.sync_copy(idx_hbm.at[pl.ds(off, _CL)], idx_v.at[b])

        def rbody(j, c):
            w16 = idx_v[b, pl.ds(j * 16, 16)]
            row_v[b, pl.ds(j * 16, 16)] = lax.shift_right_logical(w16, 3)
            return c

        lax.fori_loop(0, _CL // 16, rbody, 0)
        return pltpu.async_copy(table_hbm.at[row_v.at[b]], rows_v.at[b], sem_g)

    def extract(i, b):
        bb = jnp.full((16,), b, jnp.int32)

        def tbody(j, c):
            w16 = idx_v[b, pl.ds(j * 16, 16)]
            col = jnp.bitwise_and(w16, 7) * _D
            row16 = j * 16 + lane
            for d in range(_D):
                v = plsc.load_gather(rows_v, [bb, row16, col + d])
                planes_v[b * _D + d, pl.ds(j * 16, 16)] = v
            return c

        lax.fori_loop(0, _CL // 16, tbody, 0)
        off = chunk_off(i)
        return [
            pltpu.async_copy(
                planes_v.at[b * _D + d],
                out_hbm.at[pl.ds(d * _PLANE + off, _CL)],
                sem_w,
            )
            for d in range(_D)
        ]

    def body(p, carry):
        cs = [start_gather(_NB * p + k, k) for k in range(_NB)]
        ws = []
        for k in range(_NB):
            cs[k].wait()
            ws += extract(_NB * p + k, k)
        for w in ws:
            w.wait()
        return carry

    lax.fori_loop(0, _NCH // _NB, body, 0)


def kernel(data_index, embedding_dim, table5):
    flag = jnp.asarray(embedding_dim == _D, jnp.int32)
    idx_t = (
        data_index.T.reshape(_NT_H, 8, _NT_B, 128)
        .transpose(0, 2, 1, 3)
        .reshape(_TOTAL)
    ) * flag
    table40 = table5.reshape(_VR, _RW)
    flat = _embed_gather_t(idx_t, table40)
    o5 = flat.reshape(_D, _NT_H, _NT_B, 8, 128).transpose(1, 3, 2, 4, 0)
    return o5.reshape(_H, _B, _D).transpose(1, 0, 2)
